# baseline (speedup 1.0000x reference)
import functools

import jax
import jax.numpy as jnp
from jax import lax
from jax.experimental import pallas as pl
from jax.experimental.pallas import tpu as pltpu
from jax.experimental.pallas import tpu_sc as plsc

NC = 2
NS = 16
LANES = 16
NW = NC * NS
W = 128
CPB = 16


def kernel(tokens_batch, heads_batch, U, Ubias, V, Vbias):
    B, L = tokens_batch.shape
    N = B * L
    ED = U.shape[1]
    assert N % (NW * CPB * W) == 0
    NB = N // (NW * CPB * W)
    PW = N // NW
    IB = CPB * W

    tok = tokens_batch.reshape(-1)
    hd = heads_batch.reshape(-1)
    ub_t = Ubias.reshape(-1)
    vb_t = Vbias.reshape(-1)

    mesh = plsc.VectorSubcoreMesh(core_axis_name="c", subcore_axis_name="s")

    @functools.partial(
        pl.kernel,
        compiler_params=pltpu.CompilerParams(use_tc_tiling_on_sc=False),
        out_type=(
            jax.ShapeDtypeStruct((NW, 1, LANES), jnp.float32),
            jax.ShapeDtypeStruct((NW, LANES), jnp.float32),
        ),
        mesh=mesh,
        scratch_types=[
            pltpu.VMEM((IB,), jnp.int32),
            pltpu.VMEM((IB,), jnp.int32),
            pltpu.VMEM((2, W, ED), jnp.float32),
            pltpu.VMEM((2, W, ED), jnp.float32),
            pltpu.VMEM((2, W), jnp.float32),
            pltpu.VMEM((2, W), jnp.float32),
            pltpu.VMEM((1, LANES), jnp.float32),
            pltpu.VMEM((LANES,), jnp.float32),
            pltpu.SemaphoreType.DMA,
            pltpu.SemaphoreType.DMA,
            pltpu.SemaphoreType.DMA,
            pltpu.SemaphoreType.DMA,
            pltpu.SemaphoreType.DMA,
            pltpu.SemaphoreType.DMA,
            pltpu.SemaphoreType.DMA,
            pltpu.SemaphoreType.DMA,
        ],
    )
    def k(tok_hbm, hd_hbm, u_hbm, ub_hbm, v_hbm, vb_hbm,
          outd_hbm, outb_hbm,
          tok_i, hd_i, u_buf, v_buf, ub_buf, vb_buf, accd, accb,
          su0, su1, sv0, sv1, sb0, sb1, sc0, sc1):
        cid = lax.axis_index("c")
        sid = lax.axis_index("s")
        wid = sid * NC + cid
        base = wid * PW
        accd[...] = jnp.zeros((1, LANES), jnp.float32)
        accb[...] = jnp.zeros((LANES,), jnp.float32)
        sems_u = (su0, su1)
        sems_v = (sv0, sv1)
        sems_ub = (sb0, sb1)
        sems_vb = (sc0, sc1)

        def issue(j, g):
            pltpu.async_copy(u_hbm.at[tok_i.at[pl.ds(j * W, W)]], u_buf.at[g], sems_u[g])
            pltpu.async_copy(v_hbm.at[hd_i.at[pl.ds(j * W, W)]], v_buf.at[g], sems_v[g])
            pltpu.async_copy(ub_hbm.at[tok_i.at[pl.ds(j * W, W)]], ub_buf.at[g], sems_ub[g])
            pltpu.async_copy(vb_hbm.at[hd_i.at[pl.ds(j * W, W)]], vb_buf.at[g], sems_vb[g])

        def wait(j, g):
            pltpu.make_async_copy(u_hbm.at[tok_i.at[pl.ds(j * W, W)]], u_buf.at[g], sems_u[g]).wait()
            pltpu.make_async_copy(v_hbm.at[hd_i.at[pl.ds(j * W, W)]], v_buf.at[g], sems_v[g]).wait()
            pltpu.make_async_copy(ub_hbm.at[tok_i.at[pl.ds(j * W, W)]], ub_buf.at[g], sems_ub[g]).wait()
            pltpu.make_async_copy(vb_hbm.at[hd_i.at[pl.ds(j * W, W)]], vb_buf.at[g], sems_vb[g]).wait()

        @pl.loop(0, NB)
        def _blk(b):
            off = base + b * IB
            pltpu.sync_copy(tok_hbm.at[pl.ds(off, IB)], tok_i)
            pltpu.sync_copy(hd_hbm.at[pl.ds(off, IB)], hd_i)
            issue(0, 0)
            for j in range(CPB):
                g = j % 2
                if j + 1 < CPB:
                    issue(j + 1, 1 - g)
                wait(j, g)

                @pl.loop(0, W, step=8)
                def _rows(r):
                    t0 = u_buf[g, pl.ds(r, 1), pl.ds(0, LANES)] * v_buf[g, pl.ds(r, 1), pl.ds(0, LANES)]
                    t1 = u_buf[g, pl.ds(r, 1), pl.ds(LANES, LANES)] * v_buf[g, pl.ds(r, 1), pl.ds(LANES, LANES)]
                    t2 = u_buf[g, pl.ds(r + 1, 1), pl.ds(0, LANES)] * v_buf[g, pl.ds(r + 1, 1), pl.ds(0, LANES)]
                    t3 = u_buf[g, pl.ds(r + 1, 1), pl.ds(LANES, LANES)] * v_buf[g, pl.ds(r + 1, 1), pl.ds(LANES, LANES)]
                    for dr in range(2, 8, 2):
                        t0 = t0 + u_buf[g, pl.ds(r + dr, 1), pl.ds(0, LANES)] * v_buf[g, pl.ds(r + dr, 1), pl.ds(0, LANES)]
                        t1 = t1 + u_buf[g, pl.ds(r + dr, 1), pl.ds(LANES, LANES)] * v_buf[g, pl.ds(r + dr, 1), pl.ds(LANES, LANES)]
                        t2 = t2 + u_buf[g, pl.ds(r + dr + 1, 1), pl.ds(0, LANES)] * v_buf[g, pl.ds(r + dr + 1, 1), pl.ds(0, LANES)]
                        t3 = t3 + u_buf[g, pl.ds(r + dr + 1, 1), pl.ds(LANES, LANES)] * v_buf[g, pl.ds(r + dr + 1, 1), pl.ds(LANES, LANES)]
                    accd[...] += (t0 + t1) + (t2 + t3)

                b0 = ub_buf[g, pl.ds(0, LANES)] + vb_buf[g, pl.ds(0, LANES)]
                b1 = ub_buf[g, pl.ds(LANES, LANES)] + vb_buf[g, pl.ds(LANES, LANES)]
                b2 = ub_buf[g, pl.ds(2 * LANES, LANES)] + vb_buf[g, pl.ds(2 * LANES, LANES)]
                b3 = ub_buf[g, pl.ds(3 * LANES, LANES)] + vb_buf[g, pl.ds(3 * LANES, LANES)]
                for c2 in range(4, W // LANES, 4):
                    b0 = b0 + ub_buf[g, pl.ds(c2 * LANES, LANES)] + vb_buf[g, pl.ds(c2 * LANES, LANES)]
                    b1 = b1 + ub_buf[g, pl.ds((c2 + 1) * LANES, LANES)] + vb_buf[g, pl.ds((c2 + 1) * LANES, LANES)]
                    b2 = b2 + ub_buf[g, pl.ds((c2 + 2) * LANES, LANES)] + vb_buf[g, pl.ds((c2 + 2) * LANES, LANES)]
                    b3 = b3 + ub_buf[g, pl.ds((c2 + 3) * LANES, LANES)] + vb_buf[g, pl.ds((c2 + 3) * LANES, LANES)]
                accb[...] += (b0 + b1) + (b2 + b3)

        pltpu.sync_copy(accd, outd_hbm.at[wid])
        pltpu.sync_copy(accb, outb_hbm.at[wid])

    outd, outb = k(tok, hd, U, ub_t, V, vb_t)
    return jnp.sum(outd) + jnp.sum(outb)


# speedup vs baseline: 1.0014x; 1.0014x over previous
import functools

import jax
import jax.numpy as jnp
from jax import lax
from jax.experimental import pallas as pl
from jax.experimental.pallas import tpu as pltpu
from jax.experimental.pallas import tpu_sc as plsc

NC = 2
NS = 16
LANES = 16
NW = NC * NS
W = 128
CPB = 16


def kernel(tokens_batch, heads_batch, U, Ubias, V, Vbias):
    B, L = tokens_batch.shape
    N = B * L
    ED = U.shape[1]
    assert N % (NW * CPB * W) == 0
    NB = N // (NW * CPB * W)
    PW = N // NW
    IB = CPB * W

    tok = tokens_batch.reshape(-1)
    hd = heads_batch.reshape(-1)
    ub_t = jnp.sum(Ubias, axis=1)
    vb_t = jnp.sum(Vbias, axis=1)

    mesh = plsc.VectorSubcoreMesh(core_axis_name="c", subcore_axis_name="s")

    @functools.partial(
        pl.kernel,
        compiler_params=pltpu.CompilerParams(
            use_tc_tiling_on_sc=False, needs_layout_passes=False),
        out_type=jax.ShapeDtypeStruct((NW, 2, LANES), jnp.float32),
        mesh=mesh,
        scratch_types=[
            pltpu.VMEM((IB,), jnp.int32),
            pltpu.VMEM((IB,), jnp.int32),
            pltpu.VMEM((2, W, ED), jnp.float32),
            pltpu.VMEM((2, W, ED), jnp.float32),
            pltpu.VMEM((2, W), jnp.float32),
            pltpu.VMEM((2, W), jnp.float32),
            pltpu.VMEM((LANES,), jnp.float32),
            pltpu.VMEM((LANES,), jnp.float32),
            pltpu.SemaphoreType.DMA,
            pltpu.SemaphoreType.DMA,
            pltpu.SemaphoreType.DMA,
            pltpu.SemaphoreType.DMA,
            pltpu.SemaphoreType.DMA,
            pltpu.SemaphoreType.DMA,
            pltpu.SemaphoreType.DMA,
            pltpu.SemaphoreType.DMA,
        ],
    )
    def k(tok_hbm, hd_hbm, u_hbm, ub_hbm, v_hbm, vb_hbm,
          out_hbm,
          tok_i, hd_i, u_buf, v_buf, ub_buf, vb_buf, accd, accb,
          su0, su1, sv0, sv1, sb0, sb1, sc0, sc1):
        cid = lax.axis_index("c")
        sid = lax.axis_index("s")
        wid = sid * NC + cid
        base = wid * PW
        accd[...] = jnp.zeros((LANES,), jnp.float32)
        accb[...] = jnp.zeros((LANES,), jnp.float32)
        sems_u = (su0, su1)
        sems_v = (sv0, sv1)
        sems_ub = (sb0, sb1)
        sems_vb = (sc0, sc1)

        def issue(j, g):
            pltpu.async_copy(u_hbm.at[tok_i.at[pl.ds(j * W, W)]], u_buf.at[g], sems_u[g])
            pltpu.async_copy(v_hbm.at[hd_i.at[pl.ds(j * W, W)]], v_buf.at[g], sems_v[g])
            pltpu.async_copy(ub_hbm.at[tok_i.at[pl.ds(j * W, W)]], ub_buf.at[g], sems_ub[g])
            pltpu.async_copy(vb_hbm.at[hd_i.at[pl.ds(j * W, W)]], vb_buf.at[g], sems_vb[g])

        def wait(j, g):
            pltpu.make_async_copy(u_hbm.at[tok_i.at[pl.ds(j * W, W)]], u_buf.at[g], sems_u[g]).wait()
            pltpu.make_async_copy(v_hbm.at[hd_i.at[pl.ds(j * W, W)]], v_buf.at[g], sems_v[g]).wait()
            pltpu.make_async_copy(ub_hbm.at[tok_i.at[pl.ds(j * W, W)]], ub_buf.at[g], sems_ub[g]).wait()
            pltpu.make_async_copy(vb_hbm.at[hd_i.at[pl.ds(j * W, W)]], vb_buf.at[g], sems_vb[g]).wait()

        @pl.loop(0, NB)
        def _blk(b):
            off = base + b * IB
            pltpu.sync_copy(tok_hbm.at[pl.ds(off, IB)], tok_i)
            pltpu.sync_copy(hd_hbm.at[pl.ds(off, IB)], hd_i)
            issue(0, 0)
            for j in range(CPB):
                g = j % 2
                if j + 1 < CPB:
                    issue(j + 1, 1 - g)
                wait(j, g)

                @pl.loop(0, W, step=8)
                def _rows(r):
                    t0 = u_buf[g, r, pl.ds(0, LANES)] * v_buf[g, r, pl.ds(0, LANES)]
                    t1 = u_buf[g, r, pl.ds(LANES, LANES)] * v_buf[g, r, pl.ds(LANES, LANES)]
                    t2 = u_buf[g, r + 1, pl.ds(0, LANES)] * v_buf[g, r + 1, pl.ds(0, LANES)]
                    t3 = u_buf[g, r + 1, pl.ds(LANES, LANES)] * v_buf[g, r + 1, pl.ds(LANES, LANES)]
                    for dr in range(2, 8, 2):
                        t0 = t0 + u_buf[g, r + dr, pl.ds(0, LANES)] * v_buf[g, r + dr, pl.ds(0, LANES)]
                        t1 = t1 + u_buf[g, r + dr, pl.ds(LANES, LANES)] * v_buf[g, r + dr, pl.ds(LANES, LANES)]
                        t2 = t2 + u_buf[g, r + dr + 1, pl.ds(0, LANES)] * v_buf[g, r + dr + 1, pl.ds(0, LANES)]
                        t3 = t3 + u_buf[g, r + dr + 1, pl.ds(LANES, LANES)] * v_buf[g, r + dr + 1, pl.ds(LANES, LANES)]
                    accd[...] += (t0 + t1) + (t2 + t3)

                b0 = ub_buf[g, pl.ds(0, LANES)] + vb_buf[g, pl.ds(0, LANES)]
                b1 = ub_buf[g, pl.ds(LANES, LANES)] + vb_buf[g, pl.ds(LANES, LANES)]
                for s in range(2, W // LANES, 2):
                    b0 = b0 + ub_buf[g, pl.ds(s * LANES, LANES)] + vb_buf[g, pl.ds(s * LANES, LANES)]
                    b1 = b1 + ub_buf[g, pl.ds((s + 1) * LANES, LANES)] + vb_buf[g, pl.ds((s + 1) * LANES, LANES)]
                accb[...] += b0 + b1

        pltpu.sync_copy(accd, out_hbm.at[wid, 0])
        pltpu.sync_copy(accb, out_hbm.at[wid, 1])

    out = k(tok, hd, U, ub_t, V, vb_t)
    return jnp.sum(out)


# speedup vs baseline: 1.0016x; 1.0002x over previous
import functools

import jax
import jax.numpy as jnp
from jax import lax
from jax.experimental import pallas as pl
from jax.experimental.pallas import tpu as pltpu
from jax.experimental.pallas import tpu_sc as plsc

NC = 2
NS = 16
LANES = 16
NW = NC * NS
W = 128
CPB = 16

_SC_PARAMS = pltpu.CompilerParams(
    use_tc_tiling_on_sc=False, needs_layout_passes=False)


def kernel(tokens_batch, heads_batch, U, Ubias, V, Vbias):
    B, L = tokens_batch.shape
    N = B * L
    ED = U.shape[1]
    assert N % (NW * CPB * W) == 0
    NB = N // (NW * CPB * W)
    IB = CPB * W

    tok = tokens_batch.reshape(NW, NB, IB)
    hd = heads_batch.reshape(NW, NB, IB)
    ub_t = Ubias.reshape(-1)
    vb_t = Vbias.reshape(-1)

    mesh = plsc.VectorSubcoreMesh(core_axis_name="c", subcore_axis_name="s")

    @functools.partial(
        pl.kernel,
        compiler_params=_SC_PARAMS,
        out_type=jax.ShapeDtypeStruct((NW, LANES), jnp.float32),
        mesh=mesh,
        scratch_types=[
            pltpu.VMEM((IB,), jnp.int32),
            pltpu.VMEM((IB,), jnp.int32),
            pltpu.VMEM((2, W, ED), jnp.float32),
            pltpu.VMEM((2, W, ED), jnp.float32),
            pltpu.VMEM((LANES,), jnp.float32),
            pltpu.SemaphoreType.DMA,
            pltpu.SemaphoreType.DMA,
            pltpu.SemaphoreType.DMA,
            pltpu.SemaphoreType.DMA,
        ],
    )
    def ka(tok_hbm, hd_hbm, u_hbm, v_hbm, out_hbm,
           tok_i, hd_i, u_buf, v_buf, accd,
           su0, su1, sv0, sv1):
        cid = lax.axis_index("c")
        sid = lax.axis_index("s")
        wid = sid * NC + cid
        accd[...] = jnp.zeros((LANES,), jnp.float32)
        sems_u = (su0, su1)
        sems_v = (sv0, sv1)

        def issue(j, g):
            pltpu.async_copy(u_hbm.at[tok_i.at[pl.ds(j * W, W)]], u_buf.at[g], sems_u[g])
            pltpu.async_copy(v_hbm.at[hd_i.at[pl.ds(j * W, W)]], v_buf.at[g], sems_v[g])

        def wait(j, g):
            pltpu.make_async_copy(u_hbm.at[tok_i.at[pl.ds(j * W, W)]], u_buf.at[g], sems_u[g]).wait()
            pltpu.make_async_copy(v_hbm.at[hd_i.at[pl.ds(j * W, W)]], v_buf.at[g], sems_v[g]).wait()

        @pl.loop(0, NB)
        def _blk(b):
            pltpu.sync_copy(tok_hbm.at[wid, b], tok_i)
            pltpu.sync_copy(hd_hbm.at[wid, b], hd_i)
            issue(0, 0)
            for j in range(CPB):
                g = j % 2
                if j + 1 < CPB:
                    issue(j + 1, 1 - g)
                wait(j, g)

                @pl.loop(0, W, step=8)
                def _rows(r):
                    t0 = u_buf[g, r, pl.ds(0, LANES)] * v_buf[g, r, pl.ds(0, LANES)]
                    t1 = u_buf[g, r, pl.ds(LANES, LANES)] * v_buf[g, r, pl.ds(LANES, LANES)]
                    t2 = u_buf[g, r + 1, pl.ds(0, LANES)] * v_buf[g, r + 1, pl.ds(0, LANES)]
                    t3 = u_buf[g, r + 1, pl.ds(LANES, LANES)] * v_buf[g, r + 1, pl.ds(LANES, LANES)]
                    for dr in range(2, 8, 2):
                        t0 = t0 + u_buf[g, r + dr, pl.ds(0, LANES)] * v_buf[g, r + dr, pl.ds(0, LANES)]
                        t1 = t1 + u_buf[g, r + dr, pl.ds(LANES, LANES)] * v_buf[g, r + dr, pl.ds(LANES, LANES)]
                        t2 = t2 + u_buf[g, r + dr + 1, pl.ds(0, LANES)] * v_buf[g, r + dr + 1, pl.ds(0, LANES)]
                        t3 = t3 + u_buf[g, r + dr + 1, pl.ds(LANES, LANES)] * v_buf[g, r + dr + 1, pl.ds(LANES, LANES)]
                    accd[...] += (t0 + t1) + (t2 + t3)

        pltpu.sync_copy(accd, out_hbm.at[wid])

    @functools.partial(
        pl.kernel,
        compiler_params=_SC_PARAMS,
        out_type=jax.ShapeDtypeStruct((NW, LANES), jnp.float32),
        mesh=mesh,
        scratch_types=[
            pltpu.VMEM((IB,), jnp.int32),
            pltpu.VMEM((IB,), jnp.int32),
            pltpu.VMEM((2, W), jnp.float32),
            pltpu.VMEM((2, W), jnp.float32),
            pltpu.VMEM((LANES,), jnp.float32),
            pltpu.SemaphoreType.DMA,
            pltpu.SemaphoreType.DMA,
            pltpu.SemaphoreType.DMA,
            pltpu.SemaphoreType.DMA,
        ],
    )
    def kb(tok_hbm, hd_hbm, ub_hbm, vb_hbm, out_hbm,
           tok_i, hd_i, ub_buf, vb_buf, accb,
           su0, su1, sv0, sv1):
        cid = lax.axis_index("c")
        sid = lax.axis_index("s")
        wid = sid * NC + cid
        accb[...] = jnp.zeros((LANES,), jnp.float32)
        sems_ub = (su0, su1)
        sems_vb = (sv0, sv1)

        def issue(j, g):
            pltpu.async_copy(ub_hbm.at[tok_i.at[pl.ds(j * W, W)]], ub_buf.at[g], sems_ub[g])
            pltpu.async_copy(vb_hbm.at[hd_i.at[pl.ds(j * W, W)]], vb_buf.at[g], sems_vb[g])

        def wait(j, g):
            pltpu.make_async_copy(ub_hbm.at[tok_i.at[pl.ds(j * W, W)]], ub_buf.at[g], sems_ub[g]).wait()
            pltpu.make_async_copy(vb_hbm.at[hd_i.at[pl.ds(j * W, W)]], vb_buf.at[g], sems_vb[g]).wait()

        @pl.loop(0, NB)
        def _blk(b):
            pltpu.sync_copy(tok_hbm.at[wid, b], tok_i)
            pltpu.sync_copy(hd_hbm.at[wid, b], hd_i)
            issue(0, 0)
            for j in range(CPB):
                g = j % 2
                if j + 1 < CPB:
                    issue(j + 1, 1 - g)
                wait(j, g)
                b0 = ub_buf[g, pl.ds(0, LANES)] + vb_buf[g, pl.ds(0, LANES)]
                b1 = ub_buf[g, pl.ds(LANES, LANES)] + vb_buf[g, pl.ds(LANES, LANES)]
                for s in range(2, W // LANES, 2):
                    b0 = b0 + ub_buf[g, pl.ds(s * LANES, LANES)] + vb_buf[g, pl.ds(s * LANES, LANES)]
                    b1 = b1 + ub_buf[g, pl.ds((s + 1) * LANES, LANES)] + vb_buf[g, pl.ds((s + 1) * LANES, LANES)]
                accb[...] += b0 + b1

        pltpu.sync_copy(accb, out_hbm.at[wid])

    outa = ka(tok, hd, U, V)
    outb = kb(tok, hd, ub_t, vb_t)
    return jnp.sum(outa) + jnp.sum(outb)
